# baseline (speedup 1.0000x reference)
import functools

import jax
import jax.numpy as jnp
from jax import lax
from jax.experimental import pallas as pl
from jax.experimental.pallas import tpu as pltpu
from jax.experimental.pallas import tpu_sc as plsc

B, N, K, H = 16, 1024, 4, 128
R = 512
T = B * N
NC, NS = 2, 16
NW = NC * NS
TPW = T // NW

B_SC = 4
B_TC = B - B_SC
T_SC = B_SC * N
T_TC = B_TC * N

KS_PW = T_SC // NW
KS_CH = 8
KS_NCH = KS_PW // KS_CH

G_TOK = 32
G_ROWS = G_TOK * K
G_NCH = TPW // G_TOK


def _sc_body(idx_hbm, table_hbm, link_hbm, ent_hbm, lsum_hbm,
             idx_v, rows0, rows1, ent_v, ks0, ks1, lsum_v,
             sg0, sg1, sk0, sk1):
    wid = lax.axis_index("s") * NC + lax.axis_index("c")

    ks_tok = T_TC + wid * KS_PW
    ls_row = wid * KS_PW
    kbufs = (ks0, ks1)
    ksems = (sk0, sk1)

    def ks_start(ch, buf, sem):
        @pl.when(ch < KS_NCH)
        def _():
            pltpu.make_async_copy(
                link_hbm.at[pl.ds(ks_tok + ch * KS_CH, KS_CH)], buf, sem
            ).start()

    def ks_wait(buf, sem):
        pltpu.make_async_copy(
            link_hbm.at[pl.ds(0, KS_CH)], buf, sem).wait()

    def ks_reduce_store(ch, buf):
        def red(u, _):
            for dt in range(2):
                t = 2 * u + dt
                for j in range(R // 16):
                    s = pl.ds(j * 16, 16)
                    lsum_v[t, s] = (buf[t, 0, s] + buf[t, 1, s]
                                    + buf[t, 2, s] + buf[t, 3, s])
            return 0
        lax.fori_loop(0, KS_CH // 2, red, 0)
        pltpu.sync_copy(lsum_v, lsum_hbm.at[pl.ds(ls_row + ch * KS_CH, KS_CH)])

    ks_start(0, ks0, sk0)
    ks_start(1, ks1, sk1)

    def ks_iter(i, _):
        for b in range(2):
            ch = 2 * i + b
            ks_wait(kbufs[b], ksems[b])
            ks_reduce_store(ch, kbufs[b])
            ks_start(ch + 2, kbufs[b], ksems[b])
        return 0
    lax.fori_loop(0, KS_NCH // 2, ks_iter, 0)

    pltpu.sync_copy(idx_hbm.at[pl.ds(wid * (TPW * K // 128), TPW * K // 128)],
                    idx_v)
    gbufs = (rows0, rows1)
    gsems = (sg0, sg1)

    def g_start(ch, buf, sem):
        @pl.when(ch < G_NCH)
        def _():
            pltpu.make_async_copy(table_hbm.at[idx_v.at[ch]], buf, sem).start()

    def g_wait(buf, sem):
        pltpu.make_async_copy(
            table_hbm.at[idx_v.at[0]], buf, sem).wait()

    def g_reduce_store(ch, buf):
        def red(u, _):
            for dt in range(4):
                t = 4 * u + dt
                for j in range(H // 16):
                    s = pl.ds(j * 16, 16)
                    ent_v[t, s] = (buf[4 * t, s] + buf[4 * t + 1, s]
                                   + buf[4 * t + 2, s] + buf[4 * t + 3, s])
            return 0
        lax.fori_loop(0, G_TOK // 4, red, 0)
        pltpu.sync_copy(ent_v,
                        ent_hbm.at[pl.ds(wid * TPW + ch * G_TOK, G_TOK)])

    g_start(0, rows0, sg0)
    g_start(1, rows1, sg1)

    def g_iter(i, _):
        for b in range(2):
            ch = 2 * i + b
            g_wait(gbufs[b], gsems[b])
            g_reduce_store(ch, gbufs[b])
            g_start(ch + 2, gbufs[b], gsems[b])
        return 0
    lax.fori_loop(0, G_NCH // 2, g_iter, 0)


_sc_call = functools.partial(
    pl.kernel,
    mesh=plsc.VectorSubcoreMesh(core_axis_name="c", subcore_axis_name="s"),
    out_type=(jax.ShapeDtypeStruct((T, H), jnp.float32),
              jax.ShapeDtypeStruct((T_SC, R), jnp.float32)),
    scratch_types=[
        pltpu.VMEM((TPW * K // 128, 128), jnp.int32),
        pltpu.VMEM((G_ROWS, H), jnp.float32),
        pltpu.VMEM((G_ROWS, H), jnp.float32),
        pltpu.VMEM((G_TOK, H), jnp.float32),
        pltpu.VMEM((KS_CH, K, R), jnp.float32),
        pltpu.VMEM((KS_CH, K, R), jnp.float32),
        pltpu.VMEM((KS_CH, R), jnp.float32),
        pltpu.SemaphoreType.DMA,
        pltpu.SemaphoreType.DMA,
        pltpu.SemaphoreType.DMA,
        pltpu.SemaphoreType.DMA,
    ],
)(_sc_body)


TBLK = 1024


def _rel_body(rt_ref, fw_ref, link_ref, out_ref, w2_ref):
    @pl.when(pl.program_id(0) == 0)
    def _():
        w2_ref[...] = jnp.dot(rt_ref[...], fw_ref[...],
                              preferred_element_type=jnp.float32)

    ls = (link_ref[:, 0, :] + link_ref[:, 1, :]
          + link_ref[:, 2, :] + link_ref[:, 3, :])
    out_ref[...] = jnp.dot(ls, w2_ref[...],
                           preferred_element_type=jnp.float32
                           ).astype(jnp.bfloat16)


def _rel_sc_body(rt_ref, fw_ref, ls_ref, rel_in_ref, out_ref):
    a = jnp.dot(ls_ref[...], rt_ref[...],
                preferred_element_type=jnp.float32)
    out_ref[...] = jnp.dot(a, fw_ref[...],
                           preferred_element_type=jnp.float32
                           ).astype(jnp.bfloat16)


def _fin_body(rel_ref, ent_ref, gt_ref, g_ref, b_ref, out_ref):
    acc = rel_ref[0].astype(jnp.float32) + ent_ref[0]
    mu = jnp.mean(acc, axis=-1, keepdims=True)
    d = acc - mu
    var = jnp.mean(d * d, axis=-1, keepdims=True)
    y = d * lax.rsqrt(var + 1e-6) * g_ref[...] + b_ref[...]
    out_ref[0, 0:1, :] = gt_ref[...]
    out_ref[0, 1:, :] = y


def kernel(x, in_degree, out_degree, link, length, entity_table,
           in_deg_table, out_deg_table, rel_table, ffn_W,
           ln_gamma, ln_beta, graph_token):
    idx = x.astype(jnp.int32).reshape(T * K // 128, 128)
    link_flat = link.reshape(T, K, R)
    ent, lsum = _sc_call(idx, entity_table, link_flat)

    rel0 = pl.pallas_call(
        _rel_body,
        grid=(T_TC // TBLK,),
        in_specs=[
            pl.BlockSpec((R, H), lambda i: (0, 0)),
            pl.BlockSpec((H, H), lambda i: (0, 0)),
            pl.BlockSpec((TBLK, K, R), lambda i: (i, 0, 0)),
        ],
        out_specs=pl.BlockSpec((TBLK, H), lambda i: (i, 0)),
        out_shape=jax.ShapeDtypeStruct((T, H), jnp.bfloat16),
        scratch_shapes=[pltpu.VMEM((R, H), jnp.float32)],
    )(rel_table, ffn_W, link_flat)

    rel = pl.pallas_call(
        _rel_sc_body,
        grid=(B_SC,),
        in_specs=[
            pl.BlockSpec((R, H), lambda i: (0, 0)),
            pl.BlockSpec((H, H), lambda i: (0, 0)),
            pl.BlockSpec((N, R), lambda i: (i, 0)),
            pl.BlockSpec(memory_space=pl.ANY),
        ],
        out_specs=pl.BlockSpec((N, H), lambda i: (B_TC + i, 0)),
        out_shape=jax.ShapeDtypeStruct((T, H), jnp.bfloat16),
        input_output_aliases={3: 0},
    )(rel_table, ffn_W, lsum, rel0)

    g2 = ln_gamma.reshape(1, H)
    b2 = ln_beta.reshape(1, H)
    out = pl.pallas_call(
        _fin_body,
        grid=(B,),
        in_specs=[
            pl.BlockSpec((1, N, H), lambda i: (i, 0, 0)),
            pl.BlockSpec((1, N, H), lambda i: (i, 0, 0)),
            pl.BlockSpec((1, H), lambda i: (0, 0)),
            pl.BlockSpec((1, H), lambda i: (0, 0)),
            pl.BlockSpec((1, H), lambda i: (0, 0)),
        ],
        out_specs=pl.BlockSpec((1, N + 1, H), lambda i: (i, 0, 0)),
        out_shape=jax.ShapeDtypeStruct((B, N + 1, H), jnp.float32),
    )(rel.reshape(B, N, H), ent.reshape(B, N, H), graph_token, g2, b2)
    return out


# speedup vs baseline: 1.2621x; 1.2621x over previous
import functools

import jax
import jax.numpy as jnp
from jax import lax
from jax.experimental import pallas as pl
from jax.experimental.pallas import tpu as pltpu
from jax.experimental.pallas import tpu_sc as plsc

B, N, K, H = 16, 1024, 4, 128
R = 512
T = B * N
NC, NS = 2, 16
NW = NC * NS
TPW = T // NW

G_TOK = 32
G_ROWS = G_TOK * K
G_NCH = TPW // G_TOK


def _sc_body(idx_hbm, table_hbm, ent_hbm, idx_v, rows0, rows1, ent_v,
             sg0, sg1):
    wid = lax.axis_index("s") * NC + lax.axis_index("c")
    pltpu.sync_copy(idx_hbm.at[pl.ds(wid * (TPW * K // 128), TPW * K // 128)],
                    idx_v)
    gbufs = (rows0, rows1)
    gsems = (sg0, sg1)

    def g_start(ch, buf, sem):
        @pl.when(ch < G_NCH)
        def _():
            pltpu.make_async_copy(table_hbm.at[idx_v.at[ch]], buf, sem).start()

    def g_wait(buf, sem):
        pltpu.make_async_copy(table_hbm.at[idx_v.at[0]], buf, sem).wait()

    def g_reduce_store(ch, buf):
        def red(u, _):
            for dt in range(4):
                t = 4 * u + dt
                for j in range(H // 16):
                    s = pl.ds(j * 16, 16)
                    ent_v[t, s] = (buf[4 * t, s] + buf[4 * t + 1, s]
                                   + buf[4 * t + 2, s] + buf[4 * t + 3, s])
            return 0
        lax.fori_loop(0, G_TOK // 4, red, 0)
        pltpu.sync_copy(ent_v,
                        ent_hbm.at[pl.ds(wid * TPW + ch * G_TOK, G_TOK)])

    g_start(0, rows0, sg0)
    g_start(1, rows1, sg1)

    def g_iter(i, _):
        for b in range(2):
            ch = 2 * i + b
            g_wait(gbufs[b], gsems[b])
            g_reduce_store(ch, gbufs[b])
            g_start(ch + 2, gbufs[b], gsems[b])
        return 0
    lax.fori_loop(0, G_NCH // 2, g_iter, 0)


_sc_call = functools.partial(
    pl.kernel,
    mesh=plsc.VectorSubcoreMesh(core_axis_name="c", subcore_axis_name="s"),
    out_type=jax.ShapeDtypeStruct((T, H), jnp.float32),
    scratch_types=[
        pltpu.VMEM((TPW * K // 128, 128), jnp.int32),
        pltpu.VMEM((G_ROWS, H), jnp.float32),
        pltpu.VMEM((G_ROWS, H), jnp.float32),
        pltpu.VMEM((G_TOK, H), jnp.float32),
        pltpu.SemaphoreType.DMA,
        pltpu.SemaphoreType.DMA,
    ],
)(_sc_body)


TBLK = 1024


def _rel_body(rt_ref, fw_ref, link_ref, out_ref, w2_ref):
    @pl.when(pl.program_id(0) == 0)
    def _():
        w2_ref[...] = jnp.dot(rt_ref[...], fw_ref[...],
                              preferred_element_type=jnp.float32)

    ls = (link_ref[:, 0, :] + link_ref[:, 1, :]
          + link_ref[:, 2, :] + link_ref[:, 3, :])
    out_ref[...] = jnp.dot(ls, w2_ref[...],
                           preferred_element_type=jnp.float32
                           ).astype(jnp.bfloat16)


def _fin_body(rel_ref, ent_ref, gt_ref, g_ref, b_ref, out_ref):
    acc = rel_ref[0].astype(jnp.float32) + ent_ref[0]
    mu = jnp.mean(acc, axis=-1, keepdims=True)
    d = acc - mu
    var = jnp.mean(d * d, axis=-1, keepdims=True)
    y = d * lax.rsqrt(var + 1e-6) * g_ref[...] + b_ref[...]
    out_ref[0, 0:1, :] = gt_ref[...]
    out_ref[0, 1:, :] = y


def kernel(x, in_degree, out_degree, link, length, entity_table,
           in_deg_table, out_deg_table, rel_table, ffn_W,
           ln_gamma, ln_beta, graph_token):
    idx = x.astype(jnp.int32).reshape(T * K // 128, 128)
    link_flat = link.reshape(T, K, R)
    ent = _sc_call(idx, entity_table)

    rel = pl.pallas_call(
        _rel_body,
        grid=(T // TBLK,),
        in_specs=[
            pl.BlockSpec((R, H), lambda i: (0, 0)),
            pl.BlockSpec((H, H), lambda i: (0, 0)),
            pl.BlockSpec((TBLK, K, R), lambda i: (i, 0, 0)),
        ],
        out_specs=pl.BlockSpec((TBLK, H), lambda i: (i, 0)),
        out_shape=jax.ShapeDtypeStruct((T, H), jnp.bfloat16),
        scratch_shapes=[pltpu.VMEM((R, H), jnp.float32)],
    )(rel_table, ffn_W, link_flat)

    g2 = ln_gamma.reshape(1, H)
    b2 = ln_beta.reshape(1, H)
    out = pl.pallas_call(
        _fin_body,
        grid=(B,),
        in_specs=[
            pl.BlockSpec((1, N, H), lambda i: (i, 0, 0)),
            pl.BlockSpec((1, N, H), lambda i: (i, 0, 0)),
            pl.BlockSpec((1, H), lambda i: (0, 0)),
            pl.BlockSpec((1, H), lambda i: (0, 0)),
            pl.BlockSpec((1, H), lambda i: (0, 0)),
        ],
        out_specs=pl.BlockSpec((1, N + 1, H), lambda i: (i, 0, 0)),
        out_shape=jax.ShapeDtypeStruct((B, N + 1, H), jnp.float32),
    )(rel.reshape(B, N, H), ent.reshape(B, N, H), graph_token, g2, b2)
    return out


# speedup vs baseline: 1.4459x; 1.1457x over previous
import functools

import jax
import jax.numpy as jnp
from jax import lax
from jax.experimental import pallas as pl
from jax.experimental.pallas import tpu as pltpu
from jax.experimental.pallas import tpu_sc as plsc

B, N, K, H = 16, 1024, 4, 128
R = 512
T = B * N
NC, NS = 2, 16
NW = NC * NS
TPW = T // NW

G_TOK = 32
G_ROWS = G_TOK * K
G_NCH = TPW // G_TOK


def _sc_body(idx_hbm, table_hbm, ent_hbm, idx_v, rows0, rows1, ent_v,
             sg0, sg1):
    wid = lax.axis_index("s") * NC + lax.axis_index("c")
    pltpu.sync_copy(idx_hbm.at[pl.ds(wid * (TPW * K // 128), TPW * K // 128)],
                    idx_v)
    gbufs = (rows0, rows1)
    gsems = (sg0, sg1)

    def g_start(ch, buf, sem):
        @pl.when(ch < G_NCH)
        def _():
            pltpu.make_async_copy(table_hbm.at[idx_v.at[ch]], buf, sem).start()

    def g_wait(buf, sem):
        pltpu.make_async_copy(table_hbm.at[idx_v.at[0]], buf, sem).wait()

    def g_reduce_store(ch, buf):
        def red(u, _):
            for dt in range(4):
                t = 4 * u + dt
                for j in range(H // 16):
                    s = pl.ds(j * 16, 16)
                    ent_v[t, s] = (buf[4 * t, s] + buf[4 * t + 1, s]
                                   + buf[4 * t + 2, s] + buf[4 * t + 3, s])
            return 0
        lax.fori_loop(0, G_TOK // 4, red, 0)
        pltpu.sync_copy(ent_v,
                        ent_hbm.at[pl.ds(wid * TPW + ch * G_TOK, G_TOK)])

    g_start(0, rows0, sg0)
    g_start(1, rows1, sg1)

    def g_iter(i, _):
        for b in range(2):
            ch = 2 * i + b
            g_wait(gbufs[b], gsems[b])
            g_reduce_store(ch, gbufs[b])
            g_start(ch + 2, gbufs[b], gsems[b])
        return 0
    lax.fori_loop(0, G_NCH // 2, g_iter, 0)


_sc_call = functools.partial(
    pl.kernel,
    mesh=plsc.VectorSubcoreMesh(core_axis_name="c", subcore_axis_name="s"),
    out_type=jax.ShapeDtypeStruct((T, H), jnp.float32),
    scratch_types=[
        pltpu.VMEM((TPW * K // 128, 128), jnp.int32),
        pltpu.VMEM((G_ROWS, H), jnp.float32),
        pltpu.VMEM((G_ROWS, H), jnp.float32),
        pltpu.VMEM((G_TOK, H), jnp.float32),
        pltpu.SemaphoreType.DMA,
        pltpu.SemaphoreType.DMA,
    ],
)(_sc_body)


TBLK = 1024


def _rel_body(rt_ref, fw_ref, link_ref, out_ref, w2_ref):
    @pl.when(pl.program_id(0) == 0)
    def _():
        w2_ref[...] = jnp.dot(rt_ref[...], fw_ref[...],
                              preferred_element_type=jnp.float32)

    ls = (link_ref[:, 0, :] + link_ref[:, 1, :]
          + link_ref[:, 2, :] + link_ref[:, 3, :])
    out_ref[...] = jnp.dot(ls, w2_ref[...],
                           preferred_element_type=jnp.float32
                           ).astype(jnp.bfloat16)


FB = 8


def _fin_body(rel_ref, ent_ref, gt_ref, g_ref, b_ref, out_ref):
    acc = rel_ref[...].astype(jnp.float32) + ent_ref[...]
    mu = jnp.mean(acc, axis=-1, keepdims=True)
    d = acc - mu
    var = jnp.mean(d * d, axis=-1, keepdims=True)
    y = d * lax.rsqrt(var + 1e-6) * g_ref[...] + b_ref[...]
    out_ref[1:, :, :] = jnp.transpose(y, (1, 0, 2))
    out_ref[0:1, :, :] = jnp.broadcast_to(
        gt_ref[...].reshape(1, 1, H), (1, FB, H))


def kernel(x, in_degree, out_degree, link, length, entity_table,
           in_deg_table, out_deg_table, rel_table, ffn_W,
           ln_gamma, ln_beta, graph_token):
    idx = x.astype(jnp.int32).reshape(T * K // 128, 128)
    link_flat = link.reshape(T, K, R)
    ent = _sc_call(idx, entity_table)

    rel = pl.pallas_call(
        _rel_body,
        grid=(T // TBLK,),
        in_specs=[
            pl.BlockSpec((R, H), lambda i: (0, 0)),
            pl.BlockSpec((H, H), lambda i: (0, 0)),
            pl.BlockSpec((TBLK, K, R), lambda i: (i, 0, 0)),
        ],
        out_specs=pl.BlockSpec((TBLK, H), lambda i: (i, 0)),
        out_shape=jax.ShapeDtypeStruct((T, H), jnp.bfloat16),
        scratch_shapes=[pltpu.VMEM((R, H), jnp.float32)],
    )(rel_table, ffn_W, link_flat)

    g2 = ln_gamma.reshape(1, H)
    b2 = ln_beta.reshape(1, H)
    out_t = pl.pallas_call(
        _fin_body,
        grid=(B // FB,),
        in_specs=[
            pl.BlockSpec((FB, N, H), lambda i: (i, 0, 0)),
            pl.BlockSpec((FB, N, H), lambda i: (i, 0, 0)),
            pl.BlockSpec((1, H), lambda i: (0, 0)),
            pl.BlockSpec((1, H), lambda i: (0, 0)),
            pl.BlockSpec((1, H), lambda i: (0, 0)),
        ],
        out_specs=pl.BlockSpec((N + 1, FB, H), lambda i: (0, i, 0)),
        out_shape=jax.ShapeDtypeStruct((N + 1, B, H), jnp.float32),
    )(rel.reshape(B, N, H), ent.reshape(B, N, H), graph_token, g2, b2)
    return out_t.transpose(1, 0, 2)
